# baseline (speedup 1.0000x reference)
import jax
import jax.numpy as jnp
from jax import lax
from jax.experimental import pallas as pl
from jax.experimental.pallas import tpu as pltpu
from jax.experimental.pallas import tpu_sc as plsc

N = 100000
E = 1600000
Q = 4
QW = 8
NUM_ITER = 5

NC = 2
NS = 16
RPT = 6256
HRPT = RPT // 2
NPAD = RPT * NS
CH = 2000
CHUNKS_PER_TILE = (E // 2) // CH // NS

EB = 6400
NB = 10000

_mesh = plsc.VectorSubcoreMesh(core_axis_name="c", subcore_axis_name="s")
_sc_params = pltpu.CompilerParams(use_tc_tiling_on_sc=False)


def _scat_body(t_hbm, dst_hbm, zeros_hbm, part_hbm, table_sh, idx_v, row_v):
    c = lax.axis_index("c")
    s = lax.axis_index("s")
    r0 = s * RPT
    pltpu.sync_copy(zeros_hbm.at[pl.ds(r0, RPT)], table_sh.at[pl.ds(r0, RPT)])
    plsc.subcore_barrier()
    ebase = c * (E // 2)

    def chunk(j, carry):
        e0 = ebase + (j * NS + s) * CH
        pltpu.sync_copy(dst_hbm.at[pl.ds(e0, CH)], idx_v)
        pltpu.sync_copy(t_hbm.at[pl.ds(e0, CH)], row_v)
        pltpu.sync_copy(row_v, table_sh.at[idx_v], add=True)
        return carry

    lax.fori_loop(0, CHUNKS_PER_TILE, chunk, 0)
    plsc.subcore_barrier()
    pltpu.sync_copy(table_sh.at[pl.ds(r0, RPT)], part_hbm.at[c, pl.ds(r0, RPT)])


def _sc_scatter(t, dst, zeros):
    return pl.kernel(
        _scat_body,
        out_type=jax.ShapeDtypeStruct((NC, NPAD, QW), jnp.float32),
        mesh=_mesh,
        scratch_types=[
            pltpu.VMEM_SHARED((NPAD, QW), jnp.float32),
            pltpu.VMEM((CH,), jnp.int32),
            pltpu.VMEM((CH, QW), jnp.float32),
        ],
        compiler_params=_sc_params,
    )(t, dst, zeros)


def _gath_body(part_hbm, src_hbm, rowids_hbm, g_hbm, s_hbm,
               table_sh, idx_v, row_v, ridx_v, buf_v):
    c = lax.axis_index("c")
    s = lax.axis_index("s")
    def stage(k, carry):
        r0 = s * RPT + k * HRPT
        pltpu.sync_copy(rowids_hbm.at[pl.ds(r0, HRPT)], ridx_v)
        pltpu.sync_copy(part_hbm.at[0, pl.ds(r0, HRPT)], buf_v)
        pltpu.sync_copy(buf_v, table_sh.at[pl.ds(r0, HRPT)])
        pltpu.sync_copy(part_hbm.at[1, pl.ds(r0, HRPT)], buf_v)
        pltpu.sync_copy(buf_v, table_sh.at[ridx_v], add=True)

        @pl.when(c == 0)
        def _():
            pltpu.sync_copy(table_sh.at[pl.ds(r0, HRPT)], s_hbm.at[pl.ds(r0, HRPT)])

        return carry

    lax.fori_loop(0, 2, stage, 0)
    plsc.subcore_barrier()
    ebase = c * (E // 2)

    def chunk(j, carry):
        e0 = ebase + (j * NS + s) * CH
        pltpu.sync_copy(src_hbm.at[pl.ds(e0, CH)], idx_v)
        pltpu.sync_copy(table_sh.at[idx_v], row_v)
        pltpu.sync_copy(row_v, g_hbm.at[pl.ds(e0, CH)])
        return carry

    lax.fori_loop(0, CHUNKS_PER_TILE, chunk, 0)


def _sc_gather(part, src, rowids):
    return pl.kernel(
        _gath_body,
        out_type=(
            jax.ShapeDtypeStruct((E, QW), jnp.float32),
            jax.ShapeDtypeStruct((NPAD, QW), jnp.float32),
        ),
        mesh=_mesh,
        scratch_types=[
            pltpu.VMEM_SHARED((NPAD, QW), jnp.float32),
            pltpu.VMEM((CH,), jnp.int32),
            pltpu.VMEM((CH, QW), jnp.float32),
            pltpu.VMEM((HRPT,), jnp.int32),
            pltpu.VMEM((HRPT, QW), jnp.float32),
        ],
        compiler_params=_sc_params,
    )(part, src, rowids)


def _init_body(m_ref, eb_ref, t_ref):
    t = jnp.log1p(eb_ref[...] * m_ref[...])
    t_ref[...] = jnp.concatenate([t, jnp.zeros_like(t)], axis=1)


def _tc_log1p(msg, eb):
    return pl.pallas_call(
        _init_body,
        grid=(E // EB,),
        in_specs=[
            pl.BlockSpec((EB, Q), lambda i: (i, 0)),
            pl.BlockSpec((1, 1), lambda i: (0, 0)),
        ],
        out_specs=pl.BlockSpec((EB, QW), lambda i: (i, 0)),
        out_shape=jax.ShapeDtypeStruct((E, QW), jnp.float32),
    )(msg, eb)


def _psum_body(p_ref, o_ref):
    @pl.when(pl.program_id(0) == 0)
    def _():
        o_ref[...] = jnp.zeros_like(o_ref)

    o_ref[...] += jnp.sum(p_ref[...], axis=0, keepdims=True)


def _tc_colsum(psi):
    return pl.pallas_call(
        _psum_body,
        grid=(N // NB,),
        in_specs=[pl.BlockSpec((NB, Q), lambda i: (i, 0))],
        out_specs=pl.BlockSpec((1, Q), lambda i: (0, 0)),
        out_shape=jax.ShapeDtypeStruct((1, Q), jnp.float32),
    )(psi)


def _step_body(g_ref, trev_ref, psum_ref, hv_ref, eb_ref, m_ref, tnew_ref):
    h = hv_ref[...] * psum_ref[...]
    logits = h + g_ref[:, :Q] - trev_ref[:, :Q]
    logits = logits - jnp.max(logits, axis=1, keepdims=True)
    m = jnp.exp(logits)
    m = m / jnp.sum(m, axis=1, keepdims=True)
    m_ref[...] = m
    t = jnp.log1p(eb_ref[...] * m)
    tnew_ref[...] = jnp.concatenate([t, jnp.zeros_like(t)], axis=1)


def _tc_step(g, t, psum, hv, eb):
    nblk = E // EB
    return pl.pallas_call(
        _step_body,
        grid=(nblk,),
        in_specs=[
            pl.BlockSpec((EB, QW), lambda i: (i, 0)),
            pl.BlockSpec((EB, QW), lambda i: ((i + nblk // 2) % nblk, 0)),
            pl.BlockSpec((1, Q), lambda i: (0, 0)),
            pl.BlockSpec((1, 1), lambda i: (0, 0)),
            pl.BlockSpec((1, 1), lambda i: (0, 0)),
        ],
        out_specs=(
            pl.BlockSpec((EB, Q), lambda i: (i, 0)),
            pl.BlockSpec((EB, QW), lambda i: (i, 0)),
        ),
        out_shape=(
            jax.ShapeDtypeStruct((E, Q), jnp.float32),
            jax.ShapeDtypeStruct((E, QW), jnp.float32),
        ),
    )(g, t, psum, hv, eb)


def _psi_body(s_ref, psum_ref, hv_ref, psi_ref, po_ref):
    h = hv_ref[...] * psum_ref[...]
    logits = h + s_ref[:, :Q]
    logits = logits - jnp.max(logits, axis=1, keepdims=True)
    p = jnp.exp(logits)
    p = p / jnp.sum(p, axis=1, keepdims=True)
    psi_ref[...] = p

    @pl.when(pl.program_id(0) == 0)
    def _():
        po_ref[...] = jnp.zeros_like(po_ref)

    po_ref[...] += jnp.sum(p, axis=0, keepdims=True)


def _tc_psi(s_pad, psum, hv):
    return pl.pallas_call(
        _psi_body,
        grid=(N // NB,),
        in_specs=[
            pl.BlockSpec((NB, QW), lambda i: (i, 0)),
            pl.BlockSpec((1, Q), lambda i: (0, 0)),
            pl.BlockSpec((1, 1), lambda i: (0, 0)),
        ],
        out_specs=(
            pl.BlockSpec((NB, Q), lambda i: (i, 0)),
            pl.BlockSpec((1, Q), lambda i: (0, 0)),
        ),
        out_shape=(
            jax.ShapeDtypeStruct((N, Q), jnp.float32),
            jax.ShapeDtypeStruct((1, Q), jnp.float32),
        ),
    )(s_pad, psum, hv)


def kernel(edge_index, message_map0, marginal_psi0, beta):
    src = edge_index[0]
    dst = edge_index[1]
    mean_w = jnp.float32(E) / jnp.float32(N) / jnp.float32(N)
    eb = (jnp.exp(beta) - 1.0).astype(jnp.float32).reshape(1, 1)
    hv = (-beta * mean_w).astype(jnp.float32).reshape(1, 1)
    zeros = jnp.zeros((NPAD, QW), jnp.float32)
    rowids = jnp.arange(NPAD, dtype=jnp.int32)

    t = _tc_log1p(message_map0, eb)
    psum = _tc_colsum(marginal_psi0)

    msg = message_map0
    psi = marginal_psi0
    for _ in range(NUM_ITER):
        part = _sc_scatter(t, dst, zeros)
        g, s_pad = _sc_gather(part, src, rowids)
        psi, psum_new = _tc_psi(s_pad, psum, hv)
        msg, t = _tc_step(g, t, psum, hv, eb)
        psum = psum_new
    return (msg, psi)


# speedup vs baseline: 1.0196x; 1.0196x over previous
import jax
import jax.numpy as jnp
from jax import lax
from jax.experimental import pallas as pl
from jax.experimental.pallas import tpu as pltpu
from jax.experimental.pallas import tpu_sc as plsc

N = 100000
E = 1600000
Q = 4
QW = 8
NUM_ITER = 5

NC = 2
NS = 16
RPT = 6256
HRPT = RPT // 2
NPAD = RPT * NS
CH = 2000
CHUNKS_PER_TILE = (E // 2) // CH // NS

EB = 6400
NB = 10000

_mesh = plsc.VectorSubcoreMesh(core_axis_name="c", subcore_axis_name="s")
_sc_params = pltpu.CompilerParams(use_tc_tiling_on_sc=False)


def _scat_body(t_hbm, dst_hbm, zeros_hbm, part_hbm, table_sh, idx_v, row_v):
    c = lax.axis_index("c")
    s = lax.axis_index("s")
    r0 = s * RPT
    pltpu.sync_copy(zeros_hbm.at[pl.ds(r0, RPT)], table_sh.at[pl.ds(r0, RPT)])
    plsc.subcore_barrier()
    ebase = c * (E // 2)

    def chunk(j, carry):
        e0 = ebase + (j * NS + s) * CH
        pltpu.sync_copy(dst_hbm.at[pl.ds(e0, CH)], idx_v)
        pltpu.sync_copy(t_hbm.at[pl.ds(e0, CH)], row_v)
        pltpu.sync_copy(row_v, table_sh.at[idx_v], add=True)
        return carry

    lax.fori_loop(0, CHUNKS_PER_TILE, chunk, 0)
    plsc.subcore_barrier()
    pltpu.sync_copy(table_sh.at[pl.ds(r0, RPT)], part_hbm.at[c, pl.ds(r0, RPT)])


def _sc_scatter(t, dst, zeros):
    return pl.kernel(
        _scat_body,
        out_type=jax.ShapeDtypeStruct((NC, NPAD, QW), jnp.float32),
        mesh=_mesh,
        scratch_types=[
            pltpu.VMEM_SHARED((NPAD, QW), jnp.float32),
            pltpu.VMEM((CH,), jnp.int32),
            pltpu.VMEM((CH, QW), jnp.float32),
        ],
        compiler_params=_sc_params,
    )(t, dst, zeros)


def _gath_body(part_hbm, src_hbm, rowids_hbm, g_hbm, s_hbm,
               table_sh, idx_v, row_v, ridx_v, buf_v):
    c = lax.axis_index("c")
    s = lax.axis_index("s")
    def stage(k, carry):
        r0 = s * RPT + k * HRPT
        pltpu.sync_copy(rowids_hbm.at[pl.ds(r0, HRPT)], ridx_v)
        pltpu.sync_copy(part_hbm.at[0, pl.ds(r0, HRPT)], buf_v)
        pltpu.sync_copy(buf_v, table_sh.at[pl.ds(r0, HRPT)])
        pltpu.sync_copy(part_hbm.at[1, pl.ds(r0, HRPT)], buf_v)
        pltpu.sync_copy(buf_v, table_sh.at[ridx_v], add=True)

        @pl.when(c == 0)
        def _():
            pltpu.sync_copy(table_sh.at[pl.ds(r0, HRPT)], s_hbm.at[pl.ds(r0, HRPT)])

        return carry

    lax.fori_loop(0, 2, stage, 0)
    plsc.subcore_barrier()
    ebase = c * (E // 2)

    def chunk(j, carry):
        e0 = ebase + (j * NS + s) * CH
        pltpu.sync_copy(src_hbm.at[pl.ds(e0, CH)], idx_v)
        pltpu.sync_copy(table_sh.at[idx_v], row_v)
        pltpu.sync_copy(row_v, g_hbm.at[pl.ds(e0, CH)])
        return carry

    lax.fori_loop(0, CHUNKS_PER_TILE, chunk, 0)


def _sc_gather(part, src, rowids):
    return pl.kernel(
        _gath_body,
        out_type=(
            jax.ShapeDtypeStruct((E, QW), jnp.float32),
            jax.ShapeDtypeStruct((NPAD, QW), jnp.float32),
        ),
        mesh=_mesh,
        scratch_types=[
            pltpu.VMEM_SHARED((NPAD, QW), jnp.float32),
            pltpu.VMEM((CH,), jnp.int32),
            pltpu.VMEM((CH, QW), jnp.float32),
            pltpu.VMEM((HRPT,), jnp.int32),
            pltpu.VMEM((HRPT, QW), jnp.float32),
        ],
        compiler_params=_sc_params,
    )(part, src, rowids)


def _init_body(m_ref, eb_ref, t_ref):
    t = jnp.log1p(eb_ref[...] * m_ref[...])
    t_ref[...] = jnp.concatenate([t, jnp.zeros_like(t)], axis=1)


def _tc_log1p(msg, eb):
    return pl.pallas_call(
        _init_body,
        grid=(E // EB,),
        in_specs=[
            pl.BlockSpec((EB, Q), lambda i: (i, 0)),
            pl.BlockSpec((1, 1), lambda i: (0, 0)),
        ],
        out_specs=pl.BlockSpec((EB, QW), lambda i: (i, 0)),
        out_shape=jax.ShapeDtypeStruct((E, QW), jnp.float32),
    )(msg, eb)


def _psum_body(p_ref, o_ref):
    @pl.when(pl.program_id(0) == 0)
    def _():
        o_ref[...] = jnp.zeros_like(o_ref)

    o_ref[...] += jnp.sum(p_ref[...], axis=0, keepdims=True)


def _tc_colsum(psi):
    return pl.pallas_call(
        _psum_body,
        grid=(N // NB,),
        in_specs=[pl.BlockSpec((NB, Q), lambda i: (i, 0))],
        out_specs=pl.BlockSpec((1, Q), lambda i: (0, 0)),
        out_shape=jax.ShapeDtypeStruct((1, Q), jnp.float32),
    )(psi)


def _step_body(g_ref, trev_ref, psum_ref, hv_ref, eb_ref, m_ref, tnew_ref):
    h = hv_ref[...] * psum_ref[...]
    logits = h + g_ref[:, :Q] - trev_ref[:, :Q]
    logits = logits - jnp.max(logits, axis=1, keepdims=True)
    m = jnp.exp(logits)
    m = m / jnp.sum(m, axis=1, keepdims=True)
    m_ref[...] = m
    t = jnp.log1p(eb_ref[...] * m)
    tnew_ref[...] = jnp.concatenate([t, jnp.zeros_like(t)], axis=1)


def _tc_step(g, t, psum, hv, eb):
    nblk = E // EB
    return pl.pallas_call(
        _step_body,
        grid=(nblk,),
        in_specs=[
            pl.BlockSpec((EB, QW), lambda i: (i, 0)),
            pl.BlockSpec((EB, QW), lambda i: ((i + nblk // 2) % nblk, 0)),
            pl.BlockSpec((1, Q), lambda i: (0, 0)),
            pl.BlockSpec((1, 1), lambda i: (0, 0)),
            pl.BlockSpec((1, 1), lambda i: (0, 0)),
        ],
        out_specs=(
            pl.BlockSpec((EB, Q), lambda i: (i, 0)),
            pl.BlockSpec((EB, QW), lambda i: (i, 0)),
        ),
        out_shape=(
            jax.ShapeDtypeStruct((E, Q), jnp.float32),
            jax.ShapeDtypeStruct((E, QW), jnp.float32),
        ),
    )(g, t, psum, hv, eb)


def _last_body(g_ref, trev_ref, psum_ref, hv_ref, m_ref):
    h = hv_ref[...] * psum_ref[...]
    logits = h + g_ref[:, :Q] - trev_ref[:, :Q]
    logits = logits - jnp.max(logits, axis=1, keepdims=True)
    m = jnp.exp(logits)
    m_ref[...] = m / jnp.sum(m, axis=1, keepdims=True)


def _tc_last(g, t, psum, hv):
    nblk = E // EB
    return pl.pallas_call(
        _last_body,
        grid=(nblk,),
        in_specs=[
            pl.BlockSpec((EB, QW), lambda i: (i, 0)),
            pl.BlockSpec((EB, QW), lambda i: ((i + nblk // 2) % nblk, 0)),
            pl.BlockSpec((1, Q), lambda i: (0, 0)),
            pl.BlockSpec((1, 1), lambda i: (0, 0)),
        ],
        out_specs=pl.BlockSpec((EB, Q), lambda i: (i, 0)),
        out_shape=jax.ShapeDtypeStruct((E, Q), jnp.float32),
    )(g, t, psum, hv)


def _psi_body(s_ref, psum_ref, hv_ref, psi_ref, po_ref):
    h = hv_ref[...] * psum_ref[...]
    logits = h + s_ref[:, :Q]
    logits = logits - jnp.max(logits, axis=1, keepdims=True)
    p = jnp.exp(logits)
    p = p / jnp.sum(p, axis=1, keepdims=True)
    psi_ref[...] = p

    @pl.when(pl.program_id(0) == 0)
    def _():
        po_ref[...] = jnp.zeros_like(po_ref)

    po_ref[...] += jnp.sum(p, axis=0, keepdims=True)


def _tc_psi(s_pad, psum, hv):
    return pl.pallas_call(
        _psi_body,
        grid=(N // NB,),
        in_specs=[
            pl.BlockSpec((NB, QW), lambda i: (i, 0)),
            pl.BlockSpec((1, Q), lambda i: (0, 0)),
            pl.BlockSpec((1, 1), lambda i: (0, 0)),
        ],
        out_specs=(
            pl.BlockSpec((NB, Q), lambda i: (i, 0)),
            pl.BlockSpec((1, Q), lambda i: (0, 0)),
        ),
        out_shape=(
            jax.ShapeDtypeStruct((N, Q), jnp.float32),
            jax.ShapeDtypeStruct((1, Q), jnp.float32),
        ),
    )(s_pad, psum, hv)


def kernel(edge_index, message_map0, marginal_psi0, beta):
    src = edge_index[0]
    dst = edge_index[1]
    mean_w = jnp.float32(E) / jnp.float32(N) / jnp.float32(N)
    eb = (jnp.exp(beta) - 1.0).astype(jnp.float32).reshape(1, 1)
    hv = (-beta * mean_w).astype(jnp.float32).reshape(1, 1)
    zeros = jnp.zeros((NPAD, QW), jnp.float32)
    rowids = jnp.arange(NPAD, dtype=jnp.int32)

    t = _tc_log1p(message_map0, eb)
    psum = _tc_colsum(marginal_psi0)

    msg = message_map0
    psi = marginal_psi0
    for it in range(NUM_ITER):
        part = _sc_scatter(t, dst, zeros)
        g, s_pad = _sc_gather(part, src, rowids)
        psi, psum_new = _tc_psi(s_pad, psum, hv)
        if it + 1 < NUM_ITER:
            msg, t = _tc_step(g, t, psum, hv, eb)
        else:
            msg = _tc_last(g, t, psum, hv)
        psum = psum_new
    return (msg, psi)


# speedup vs baseline: 1.0770x; 1.0563x over previous
import jax
import jax.numpy as jnp
from jax import lax
from jax.experimental import pallas as pl
from jax.experimental.pallas import tpu as pltpu
from jax.experimental.pallas import tpu_sc as plsc

N = 100000
E = 1600000
Q = 4
QW = 8
NUM_ITER = 5

NC = 2
NS = 16
RPT = 6256
HRPT = RPT // 2
NPAD = RPT * NS
CH = 2000
CHUNKS_PER_TILE = (E // 2) // CH // NS

EB = 6400
NB = 10000

_mesh = plsc.VectorSubcoreMesh(core_axis_name="c", subcore_axis_name="s")
_sc_params = pltpu.CompilerParams(use_tc_tiling_on_sc=False)


def _scat_body(t_hbm, dst_hbm, zeros_hbm, part_hbm, table_sh, idx_v, row_v):
    c = lax.axis_index("c")
    s = lax.axis_index("s")
    r0 = s * RPT
    pltpu.sync_copy(zeros_hbm.at[pl.ds(r0, RPT)], table_sh.at[pl.ds(r0, RPT)])
    plsc.subcore_barrier()
    ebase = c * (E // 2)

    def chunk(j, carry):
        e0 = ebase + (j * NS + s) * CH
        pltpu.sync_copy(dst_hbm.at[pl.ds(e0, CH)], idx_v)
        pltpu.sync_copy(t_hbm.at[pl.ds(e0, CH)], row_v)
        pltpu.sync_copy(row_v, table_sh.at[idx_v], add=True)
        return carry

    lax.fori_loop(0, CHUNKS_PER_TILE, chunk, 0)
    plsc.subcore_barrier()
    pltpu.sync_copy(table_sh.at[pl.ds(r0, RPT)], part_hbm.at[c, pl.ds(r0, RPT)])


def _sc_scatter(t, dst, zeros):
    return pl.kernel(
        _scat_body,
        out_type=jax.ShapeDtypeStruct((NC, NPAD, QW), jnp.float32),
        mesh=_mesh,
        scratch_types=[
            pltpu.VMEM_SHARED((NPAD, QW), jnp.float32),
            pltpu.VMEM((CH,), jnp.int32),
            pltpu.VMEM((CH, QW), jnp.float32),
        ],
        compiler_params=_sc_params,
    )(t, dst, zeros)


def _gath_body(part_hbm, src_hbm, rowids_hbm, g_hbm, s_hbm,
               table_sh, idx_v, row_v, ridx_v, buf_v):
    c = lax.axis_index("c")
    s = lax.axis_index("s")
    def stage(k, carry):
        r0 = s * RPT + k * HRPT
        pltpu.sync_copy(rowids_hbm.at[pl.ds(r0, HRPT)], ridx_v)
        pltpu.sync_copy(part_hbm.at[0, pl.ds(r0, HRPT)], buf_v)
        pltpu.sync_copy(buf_v, table_sh.at[pl.ds(r0, HRPT)])
        pltpu.sync_copy(part_hbm.at[1, pl.ds(r0, HRPT)], buf_v)
        pltpu.sync_copy(buf_v, table_sh.at[ridx_v], add=True)

        @pl.when(c == 0)
        def _():
            pltpu.sync_copy(table_sh.at[pl.ds(r0, HRPT)], s_hbm.at[pl.ds(r0, HRPT)])

        return carry

    lax.fori_loop(0, 2, stage, 0)
    plsc.subcore_barrier()
    ebase = c * (E // 2)

    def chunk(j, carry):
        e0 = ebase + (j * NS + s) * CH
        pltpu.sync_copy(src_hbm.at[pl.ds(e0, CH)], idx_v)
        pltpu.sync_copy(table_sh.at[idx_v], row_v)
        pltpu.sync_copy(row_v, g_hbm.at[pl.ds(e0, CH)])
        return carry

    lax.fori_loop(0, CHUNKS_PER_TILE, chunk, 0)


def _sc_gather(part, src, rowids):
    return pl.kernel(
        _gath_body,
        out_type=(
            jax.ShapeDtypeStruct((E, QW), jnp.float32),
            jax.ShapeDtypeStruct((NPAD, QW), jnp.float32),
        ),
        mesh=_mesh,
        scratch_types=[
            pltpu.VMEM_SHARED((NPAD, QW), jnp.float32),
            pltpu.VMEM((CH,), jnp.int32),
            pltpu.VMEM((CH, QW), jnp.float32),
            pltpu.VMEM((HRPT,), jnp.int32),
            pltpu.VMEM((HRPT, QW), jnp.float32),
        ],
        compiler_params=_sc_params,
    )(part, src, rowids)


def _init_body(m_ref, eb_ref, t_ref):
    t = jnp.log1p(eb_ref[...] * m_ref[...])
    t_ref[...] = jnp.concatenate([t, jnp.zeros_like(t)], axis=1)


def _tc_log1p(msg, eb):
    return pl.pallas_call(
        _init_body,
        grid=(E // EB,),
        in_specs=[
            pl.BlockSpec((EB, Q), lambda i: (i, 0)),
            pl.BlockSpec((1, 1), lambda i: (0, 0)),
        ],
        out_specs=pl.BlockSpec((EB, QW), lambda i: (i, 0)),
        out_shape=jax.ShapeDtypeStruct((E, QW), jnp.float32),
    )(msg, eb)


def _psum_body(p_ref, o_ref):
    @pl.when(pl.program_id(0) == 0)
    def _():
        o_ref[...] = jnp.zeros_like(o_ref)

    o_ref[...] += jnp.sum(p_ref[...], axis=0, keepdims=True)


def _tc_colsum(psi):
    return pl.pallas_call(
        _psum_body,
        grid=(N // NB,),
        in_specs=[pl.BlockSpec((NB, Q), lambda i: (i, 0))],
        out_specs=pl.BlockSpec((1, Q), lambda i: (0, 0)),
        out_shape=jax.ShapeDtypeStruct((1, Q), jnp.float32),
    )(psi)


def _step_body(g_ref, trev_ref, psum_ref, hv_ref, eb_ref, tnew_ref):
    h = hv_ref[...] * psum_ref[...]
    logits = h + g_ref[:, :Q] - trev_ref[:, :Q]
    logits = logits - jnp.max(logits, axis=1, keepdims=True)
    m = jnp.exp(logits)
    m = m / jnp.sum(m, axis=1, keepdims=True)
    t = jnp.log1p(eb_ref[...] * m)
    tnew_ref[...] = jnp.concatenate([t, jnp.zeros_like(t)], axis=1)


def _tc_step(g, t, psum, hv, eb):
    nblk = E // EB
    return pl.pallas_call(
        _step_body,
        grid=(nblk,),
        in_specs=[
            pl.BlockSpec((EB, QW), lambda i: (i, 0)),
            pl.BlockSpec((EB, QW), lambda i: ((i + nblk // 2) % nblk, 0)),
            pl.BlockSpec((1, Q), lambda i: (0, 0)),
            pl.BlockSpec((1, 1), lambda i: (0, 0)),
            pl.BlockSpec((1, 1), lambda i: (0, 0)),
        ],
        out_specs=pl.BlockSpec((EB, QW), lambda i: (i, 0)),
        out_shape=jax.ShapeDtypeStruct((E, QW), jnp.float32),
    )(g, t, psum, hv, eb)


def _last_body(g_ref, trev_ref, psum_ref, hv_ref, m_ref):
    h = hv_ref[...] * psum_ref[...]
    logits = h + g_ref[:, :Q] - trev_ref[:, :Q]
    logits = logits - jnp.max(logits, axis=1, keepdims=True)
    m = jnp.exp(logits)
    m_ref[...] = m / jnp.sum(m, axis=1, keepdims=True)


def _tc_last(g, t, psum, hv):
    nblk = E // EB
    return pl.pallas_call(
        _last_body,
        grid=(nblk,),
        in_specs=[
            pl.BlockSpec((EB, QW), lambda i: (i, 0)),
            pl.BlockSpec((EB, QW), lambda i: ((i + nblk // 2) % nblk, 0)),
            pl.BlockSpec((1, Q), lambda i: (0, 0)),
            pl.BlockSpec((1, 1), lambda i: (0, 0)),
        ],
        out_specs=pl.BlockSpec((EB, Q), lambda i: (i, 0)),
        out_shape=jax.ShapeDtypeStruct((E, Q), jnp.float32),
    )(g, t, psum, hv)


def _psi_body(s_ref, psum_ref, hv_ref, psi_ref, po_ref):
    h = hv_ref[...] * psum_ref[...]
    logits = h + s_ref[:, :Q]
    logits = logits - jnp.max(logits, axis=1, keepdims=True)
    p = jnp.exp(logits)
    p = p / jnp.sum(p, axis=1, keepdims=True)
    psi_ref[...] = p

    @pl.when(pl.program_id(0) == 0)
    def _():
        po_ref[...] = jnp.zeros_like(po_ref)

    po_ref[...] += jnp.sum(p, axis=0, keepdims=True)


def _tc_psi(s_pad, psum, hv):
    return pl.pallas_call(
        _psi_body,
        grid=(N // NB,),
        in_specs=[
            pl.BlockSpec((NB, QW), lambda i: (i, 0)),
            pl.BlockSpec((1, Q), lambda i: (0, 0)),
            pl.BlockSpec((1, 1), lambda i: (0, 0)),
        ],
        out_specs=(
            pl.BlockSpec((NB, Q), lambda i: (i, 0)),
            pl.BlockSpec((1, Q), lambda i: (0, 0)),
        ),
        out_shape=(
            jax.ShapeDtypeStruct((N, Q), jnp.float32),
            jax.ShapeDtypeStruct((1, Q), jnp.float32),
        ),
    )(s_pad, psum, hv)


def kernel(edge_index, message_map0, marginal_psi0, beta):
    src = edge_index[0]
    dst = edge_index[1]
    mean_w = jnp.float32(E) / jnp.float32(N) / jnp.float32(N)
    eb = (jnp.exp(beta) - 1.0).astype(jnp.float32).reshape(1, 1)
    hv = (-beta * mean_w).astype(jnp.float32).reshape(1, 1)
    zeros = jnp.zeros((NPAD, QW), jnp.float32)
    rowids = jnp.arange(NPAD, dtype=jnp.int32)

    t = _tc_log1p(message_map0, eb)
    psum = _tc_colsum(marginal_psi0)

    msg = message_map0
    psi = marginal_psi0
    for it in range(NUM_ITER):
        part = _sc_scatter(t, dst, zeros)
        g, s_pad = _sc_gather(part, src, rowids)
        psi, psum_new = _tc_psi(s_pad, psum, hv)
        if it + 1 < NUM_ITER:
            t = _tc_step(g, t, psum, hv, eb)
        else:
            msg = _tc_last(g, t, psum, hv)
        psum = psum_new
    return (msg, psi)
